# baseline (speedup 1.0000x reference)
import functools

import jax
import jax.numpy as jnp
from jax import lax
from jax.experimental import pallas as pl
from jax.experimental.pallas import tpu as pltpu
from jax.experimental.pallas import tpu_sc as plsc

NC = 2
NS = 16
NW = NC * NS
DEGW = 16

_mesh = lambda: plsc.VectorSubcoreMesh(core_axis_name="c", subcore_axis_name="s")


def _make_deg(n_pad, ch):
  rpt = n_pad // NS

  @functools.partial(
      pl.kernel,
      out_type=jax.ShapeDtypeStruct((NC, n_pad, DEGW), jnp.float32),
      mesh=_mesh(),
      compiler_params=pltpu.CompilerParams(use_tc_tiling_on_sc=False),
      scratch_types=[
          pltpu.VMEM((ch, 128), jnp.int32),
          pltpu.VMEM((128, DEGW), jnp.float32),
          pltpu.VMEM_SHARED((n_pad, DEGW), jnp.float32),
      ],
  )
  def deg_kernel(dst2_hbm, zeros_hbm, ones_hbm, out_hbm, dst_v, ones_v, dacc):
    cid = lax.axis_index("c")
    sid = lax.axis_index("s")
    wid = cid * NS + sid
    pltpu.sync_copy(dst2_hbm.at[pl.ds(wid * ch, ch)], dst_v)
    pltpu.sync_copy(ones_hbm, ones_v)
    pltpu.sync_copy(zeros_hbm.at[pl.ds(sid * rpt, rpt)],
                    dacc.at[pl.ds(sid * rpt, rpt)])
    plsc.subcore_barrier()

    def body(j, carry):
      pltpu.sync_copy(ones_v, dacc.at[dst_v.at[j]], add=True)
      return carry

    lax.fori_loop(0, ch, body, 0)
    plsc.subcore_barrier()
    pltpu.sync_copy(dacc.at[pl.ds(sid * rpt, rpt)],
                    out_hbm.at[cid, pl.ds(sid * rpt, rpt)])

  return deg_kernel


def _make_agg(n_pad, emb, cht):
  rpt = n_pad // NS
  hw = emb // 2

  @functools.partial(
      pl.kernel,
      out_type=jax.ShapeDtypeStruct((n_pad, emb), jnp.float32),
      mesh=_mesh(),
      compiler_params=pltpu.CompilerParams(use_tc_tiling_on_sc=False),
      scratch_types=[
          pltpu.VMEM((4, 128), jnp.int32),
          pltpu.VMEM((4, 128), jnp.int32),
          pltpu.VMEM((128, hw), jnp.float32),
          pltpu.VMEM((128, hw), jnp.float32),
          pltpu.VMEM((128, hw), jnp.float32),
          pltpu.VMEM((128, hw), jnp.float32),
          pltpu.SemaphoreType.DMA,
          pltpu.SemaphoreType.DMA,
          pltpu.SemaphoreType.DMA,
          pltpu.SemaphoreType.DMA,
          pltpu.SemaphoreType.DMA,
          pltpu.SemaphoreType.DMA,
          pltpu.SemaphoreType.DMA,
          pltpu.SemaphoreType.DMA,
          pltpu.SemaphoreType.DMA,
          pltpu.SemaphoreType.DMA,
          pltpu.SemaphoreType.DMA,
          pltpu.SemaphoreType.DMA,
          pltpu.VMEM_SHARED((n_pad, hw), jnp.float32),
          pltpu.VMEM_SHARED((n_pad, hw), jnp.float32),
      ],
  )
  def agg_kernel(tab_hbm, src1_hbm, dst1_hbm, zeros_hbm, out_hbm,
                 si4, di4, r0, r1, r2, r3,
                 i0, i1, i2, i3, g0, g1, g2, g3, s0, s1, s2, s3,
                 tabs, acc):
    rows = (r0, r1, r2, r3)
    isem = (i0, i1, i2, i3)
    gsem = (g0, g1, g2, g3)
    ssem = (s0, s1, s2, s3)
    cid = lax.axis_index("c")
    sid = lax.axis_index("s")
    base = sid * cht * 128
    col = cid * hw
    pltpu.sync_copy(tab_hbm.at[pl.ds(sid * rpt, rpt), pl.ds(col, hw)],
                    tabs.at[pl.ds(sid * rpt, rpt)])
    pltpu.sync_copy(zeros_hbm, r0)
    for zb in range(0, rpt, 128):
      zn = min(128, rpt - zb)
      pltpu.sync_copy(r0.at[pl.ds(0, zn)],
                      acc.at[pl.ds(sid * rpt + zb, zn)])

    def istage(j, t4):
      pltpu.async_copy(src1_hbm.at[pl.ds(base + j * 128, 128)],
                       si4.at[t4], isem[t4])
      pltpu.async_copy(dst1_hbm.at[pl.ds(base + j * 128, 128)],
                       di4.at[t4], isem[t4])

    def iwait(t4):
      pltpu.make_async_copy(src1_hbm.at[pl.ds(0, 128)],
                            si4.at[t4], isem[t4]).wait()
      pltpu.make_async_copy(dst1_hbm.at[pl.ds(0, 128)],
                            di4.at[t4], isem[t4]).wait()

    def gstart(t4):
      pltpu.async_copy(tabs.at[si4.at[t4]], rows[t4], gsem[t4])

    def gwait(t4):
      pltpu.make_async_copy(tabs.at[si4.at[0]], rows[t4], gsem[t4]).wait()

    def sstart(t4):
      pltpu.async_copy(rows[t4], acc.at[di4.at[t4]], ssem[t4], add=True)

    def swait(t4):
      pltpu.make_async_copy(rows[t4], acc.at[di4.at[0]], ssem[t4]).wait()

    istage(0, 0)
    istage(1, 1)
    plsc.subcore_barrier()
    iwait(0)
    gstart(0)

    def body(jj, carry):
      for t in range(4):
        j = jj * 4 + t
        f4 = (t + 2) % 4
        n4 = (t + 1) % 4

        if t < 2:
          @pl.when(j >= 2)
          def _():
            swait(f4)
        else:
          swait(f4)

        @pl.when(j + 2 < cht)
        def _():
          istage(j + 2, f4)

        @pl.when(j + 1 < cht)
        def _():
          iwait(n4)
          gstart(n4)

        gwait(t)
        sstart(t)
      return carry

    lax.fori_loop(0, cht // 4, body, 0)
    swait(2)
    swait(3)
    plsc.subcore_barrier()
    pltpu.sync_copy(acc.at[pl.ds(sid * rpt, rpt)],
                    out_hbm.at[pl.ds(sid * rpt, rpt), pl.ds(col, hw)])

  return agg_kernel


def _make_pair_gather(n_pad, emb, bpt):
  @functools.partial(
      pl.kernel,
      out_type=(
          jax.ShapeDtypeStruct((NW * bpt * 128, emb), jnp.float32),
          jax.ShapeDtypeStruct((NW * bpt * 128, emb), jnp.float32),
      ),
      mesh=_mesh(),
      scratch_types=[
          pltpu.VMEM((bpt * 128,), jnp.int32),
          pltpu.VMEM((bpt * 128,), jnp.int32),
          pltpu.VMEM((128, emb), jnp.float32),
      ],
  )
  def gather_kernel(t_hbm, mid1_hbm, uid1_hbm, outm_hbm, outu_hbm,
                    mid_v, uid_v, rows_v):
    cid = lax.axis_index("c")
    sid = lax.axis_index("s")
    wid = cid * NS + sid
    pltpu.sync_copy(mid1_hbm.at[pl.ds(wid * bpt * 128, bpt * 128)], mid_v)
    pltpu.sync_copy(uid1_hbm.at[pl.ds(wid * bpt * 128, bpt * 128)], uid_v)

    def body(j, carry):
      base = wid * bpt * 128 + j * 128
      pltpu.sync_copy(t_hbm.at[mid_v.at[pl.ds(j * 128, 128)]], rows_v)
      pltpu.sync_copy(rows_v, outm_hbm.at[pl.ds(base, 128)])
      pltpu.sync_copy(t_hbm.at[uid_v.at[pl.ds(j * 128, 128)]], rows_v)
      pltpu.sync_copy(rows_v, outu_hbm.at[pl.ds(base, 128)])
      return carry

    lax.fori_loop(0, bpt, body, 0)

  return gather_kernel


def _make_tc1(n_movies, n_users, n_pad, emb):
  n = n_movies + n_users

  def tc1_body(me_ref, ue_ref, w1t_ref, d0_ref, d1_ref, g_ref, wgt_ref,
               fb1_ref, dis_ref, h1p_ref, gp_ref):
    deg = d0_ref[...] + d1_ref[...] + 1.0
    dis = lax.rsqrt(deg)
    dis_ref[...] = dis
    h1p_ref[0:n_movies, :] = dis[0:n_movies] * jnp.dot(
        me_ref[...], w1t_ref[...], preferred_element_type=jnp.float32)
    h1p_ref[n_movies:n, :] = dis[n_movies:n] * jnp.dot(
        ue_ref[...], w1t_ref[...], preferred_element_type=jnp.float32)
    h1p_ref[n:n_pad, :] = jnp.zeros((n_pad - n, emb), jnp.float32)
    gp_ref[...] = jnp.dot(g_ref[...], wgt_ref[...],
                          preferred_element_type=jnp.float32) + fb1_ref[...]

  return tc1_body


def _tc2_body(a_ref, h1p_ref, dis_ref, b1_ref, w2t_ref, h2p_ref):
  s = a_ref[...] + h1p_ref[...]
  out1 = jnp.maximum(dis_ref[...] * s + b1_ref[...], 0.0)
  h2p_ref[...] = dis_ref[...] * jnp.dot(out1, w2t_ref[...],
                                        preferred_element_type=jnp.float32)


def _make_tc3(n_movies, n_users, n_pad, emb):
  n = n_movies + n_users

  def tc3_body(a_ref, h2p_ref, dis_ref, b2_ref, wmt_ref, wut_ref, t_ref):
    out2 = dis_ref[...] * (a_ref[...] + h2p_ref[...]) + b2_ref[...]
    t_ref[0:n_movies, :] = jnp.dot(
        out2[0:n_movies, :], wmt_ref[...], preferred_element_type=jnp.float32)
    t_ref[n_movies:n, :] = jnp.dot(
        out2[n_movies:n, :], wut_ref[...], preferred_element_type=jnp.float32)
    t_ref[n:n_pad, :] = jnp.zeros((n_pad - n, emb), jnp.float32)

  return tc3_body


def _tc4_body(rm_ref, ru_ref, gp_ref, w2_ref, fb2_ref, out_ref):
  fc1 = jnp.maximum(rm_ref[...] + ru_ref[...] + gp_ref[...], 0.0)
  out_ref[...] = jnp.sum(fc1 * w2_ref[...], axis=1, keepdims=True) + fb2_ref[...]


def kernel(movie_id, user_id, genre_id, edge_index, movie_emb, user_emb,
           W1, b1, W2, b2, fcW1, fcb1, fcW2, fcb2):
  f32, i32 = jnp.float32, jnp.int32
  n_movies, emb = movie_emb.shape
  n_users = user_emb.shape[0]
  n = n_movies + n_users
  hid = W1.shape[0]
  bsz = movie_id.shape[0]
  e = edge_index.shape[1]

  n_pad = ((n + 1 + 127) // 128) * 128
  ch = (e + NW * 128 - 1) // (NW * 128)
  ch = ((ch + 7) // 8) * 8
  e_pad = NW * ch * 128
  bpt = bsz // (NW * 128)

  src = edge_index[0].astype(i32)
  dst = edge_index[1].astype(i32)
  pad_src = jnp.full((e_pad - e,), n, i32)
  pad_dst = n + jnp.arange(e_pad - e, dtype=i32) % (n_pad - n)
  src1 = jnp.concatenate([src, pad_src])
  dst1 = jnp.concatenate([dst, pad_dst])
  dst2 = dst1.reshape(NW * ch, 128)

  zeros_deg = jnp.zeros((n_pad, DEGW), f32)
  zeros_blk = jnp.zeros((128, emb // 2), f32)
  ones16 = jnp.ones((128, DEGW), f32)

  w1t = W1.T.astype(f32)
  w2t = W2.T.astype(f32)
  wmt = fcW1[:, :emb].T.astype(f32)
  wut = fcW1[:, emb:2 * emb].T.astype(f32)
  wgt = fcW1[:, 2 * emb:].T.astype(f32)
  b1r = b1.reshape(1, hid).astype(f32)
  b2r = b2.reshape(1, hid).astype(f32)
  fb1 = fcb1.reshape(1, hid).astype(f32)
  w2row = fcW2.reshape(1, hid).astype(f32)
  fb2 = fcb2.reshape(1, 1).astype(f32)
  genre = genre_id.astype(f32)

  mid1 = movie_id.astype(i32)
  uid1 = user_id.astype(i32) + n_movies

  deg_parts = _make_deg(n_pad, ch)(dst2, zeros_deg, ones16)
  d0 = deg_parts[0, :, 0:1]
  d1 = deg_parts[1, :, 0:1]

  dis, h1p, gproj = pl.pallas_call(
      _make_tc1(n_movies, n_users, n_pad, emb),
      out_shape=(
          jax.ShapeDtypeStruct((n_pad, 1), f32),
          jax.ShapeDtypeStruct((n_pad, emb), f32),
          jax.ShapeDtypeStruct((bsz, hid), f32),
      ),
  )(movie_emb.astype(f32), user_emb.astype(f32), w1t, d0, d1,
    genre, wgt, fb1)

  cht = e_pad // (NS * 128)
  agg = _make_agg(n_pad, emb, cht)

  agg1 = agg(h1p, src1, dst1, zeros_blk)

  h2p = pl.pallas_call(
      _tc2_body,
      out_shape=jax.ShapeDtypeStruct((n_pad, emb), f32),
  )(agg1, h1p, dis, b1r, w2t)

  agg2 = agg(h2p, src1, dst1, zeros_blk)

  t_tab = pl.pallas_call(
      _make_tc3(n_movies, n_users, n_pad, emb),
      out_shape=jax.ShapeDtypeStruct((n_pad, emb), f32),
  )(agg2, h2p, dis, b2r, wmt, wut)

  rows_m, rows_u = _make_pair_gather(n_pad, emb, bpt)(t_tab, mid1, uid1)

  out = pl.pallas_call(
      _tc4_body,
      out_shape=jax.ShapeDtypeStruct((bsz, 1), f32),
  )(rows_m, rows_u, gproj, w2row, fb2)
  return out


# speedup vs baseline: 1.0170x; 1.0170x over previous
import functools

import jax
import jax.numpy as jnp
from jax import lax
from jax.experimental import pallas as pl
from jax.experimental.pallas import tpu as pltpu
from jax.experimental.pallas import tpu_sc as plsc

NC = 2
NS = 16
NW = NC * NS
DEGW = 16

_mesh = lambda: plsc.VectorSubcoreMesh(core_axis_name="c", subcore_axis_name="s")


def _make_deg(n_pad, ch):
  rpt = n_pad // NS

  @functools.partial(
      pl.kernel,
      out_type=jax.ShapeDtypeStruct((NC, n_pad, DEGW), jnp.float32),
      mesh=_mesh(),
      compiler_params=pltpu.CompilerParams(use_tc_tiling_on_sc=False),
      scratch_types=[
          pltpu.VMEM((ch, 128), jnp.int32),
          pltpu.VMEM((128, DEGW), jnp.float32),
          pltpu.VMEM_SHARED((n_pad, DEGW), jnp.float32),
      ],
  )
  def deg_kernel(dst2_hbm, zeros_hbm, ones_hbm, out_hbm, dst_v, ones_v, dacc):
    cid = lax.axis_index("c")
    sid = lax.axis_index("s")
    wid = cid * NS + sid
    pltpu.sync_copy(dst2_hbm.at[pl.ds(wid * ch, ch)], dst_v)
    pltpu.sync_copy(ones_hbm, ones_v)
    pltpu.sync_copy(zeros_hbm.at[pl.ds(sid * rpt, rpt)],
                    dacc.at[pl.ds(sid * rpt, rpt)])
    plsc.subcore_barrier()

    def body(j, carry):
      pltpu.sync_copy(ones_v, dacc.at[dst_v.at[j]], add=True)
      return carry

    lax.fori_loop(0, ch, body, 0)
    plsc.subcore_barrier()
    pltpu.sync_copy(dacc.at[pl.ds(sid * rpt, rpt)],
                    out_hbm.at[cid, pl.ds(sid * rpt, rpt)])

  return deg_kernel


def _make_agg(n_pad, emb, cht):
  rpt = n_pad // NS
  hw = emb // 2

  @functools.partial(
      pl.kernel,
      out_type=jax.ShapeDtypeStruct((n_pad, emb), jnp.float32),
      mesh=_mesh(),
      compiler_params=pltpu.CompilerParams(use_tc_tiling_on_sc=False),
      scratch_types=[
          pltpu.VMEM((4, 128), jnp.int32),
          pltpu.VMEM((4, 128), jnp.int32),
          pltpu.VMEM((128, hw), jnp.float32),
          pltpu.VMEM((128, hw), jnp.float32),
          pltpu.VMEM((128, hw), jnp.float32),
          pltpu.VMEM((128, hw), jnp.float32),
          pltpu.SemaphoreType.DMA,
          pltpu.SemaphoreType.DMA,
          pltpu.SemaphoreType.DMA,
          pltpu.SemaphoreType.DMA,
          pltpu.SemaphoreType.DMA,
          pltpu.SemaphoreType.DMA,
          pltpu.SemaphoreType.DMA,
          pltpu.SemaphoreType.DMA,
          pltpu.SemaphoreType.DMA,
          pltpu.SemaphoreType.DMA,
          pltpu.SemaphoreType.DMA,
          pltpu.SemaphoreType.DMA,
          pltpu.VMEM_SHARED((n_pad, hw), jnp.float32),
          pltpu.VMEM_SHARED((n_pad, hw), jnp.float32),
      ],
  )
  def agg_kernel(tab_hbm, src1_hbm, dst1_hbm, zeros_hbm, out_hbm,
                 si4, di4, r0, r1, r2, r3,
                 i0, i1, i2, i3, g0, g1, g2, g3, s0, s1, s2, s3,
                 tabs, acc):
    rows = (r0, r1, r2, r3)
    isem = (i0, i1, i2, i3)
    gsem = (g0, g1, g2, g3)
    ssem = (s0, s1, s2, s3)
    cid = lax.axis_index("c")
    sid = lax.axis_index("s")
    base = sid * cht * 128
    col = cid * hw
    pltpu.sync_copy(tab_hbm.at[pl.ds(sid * rpt, rpt), pl.ds(col, hw)],
                    tabs.at[pl.ds(sid * rpt, rpt)])
    pltpu.sync_copy(zeros_hbm, r0)
    for zb in range(0, rpt, 128):
      zn = min(128, rpt - zb)
      pltpu.sync_copy(r0.at[pl.ds(0, zn)],
                      acc.at[pl.ds(sid * rpt + zb, zn)])

    def istage(j, t4):
      pltpu.async_copy(src1_hbm.at[pl.ds(base + j * 128, 128)],
                       si4.at[t4], isem[t4])
      pltpu.async_copy(dst1_hbm.at[pl.ds(base + j * 128, 128)],
                       di4.at[t4], isem[t4])

    def iwait(t4):
      pltpu.make_async_copy(src1_hbm.at[pl.ds(0, 128)],
                            si4.at[t4], isem[t4]).wait()
      pltpu.make_async_copy(dst1_hbm.at[pl.ds(0, 128)],
                            di4.at[t4], isem[t4]).wait()

    def gstart(t4):
      pltpu.async_copy(tabs.at[si4.at[t4]], rows[t4], gsem[t4])

    def gwait(t4):
      pltpu.make_async_copy(tabs.at[si4.at[0]], rows[t4], gsem[t4]).wait()

    def sstart(t4):
      pltpu.async_copy(rows[t4], acc.at[di4.at[t4]], ssem[t4], add=True)

    def swait(t4):
      pltpu.make_async_copy(rows[t4], acc.at[di4.at[0]], ssem[t4]).wait()

    istage(0, 0)
    istage(1, 1)
    plsc.subcore_barrier()
    iwait(0)
    gstart(0)

    def body(jj, carry):
      for t in range(4):
        j = jj * 4 + t
        f4 = (t + 2) % 4
        n4 = (t + 1) % 4

        if t < 2:
          @pl.when(j >= 2)
          def _():
            swait(f4)
        else:
          swait(f4)

        @pl.when(j + 2 < cht)
        def _():
          istage(j + 2, f4)

        @pl.when(j + 1 < cht)
        def _():
          iwait(n4)
          gstart(n4)

        gwait(t)
        sstart(t)
      return carry

    lax.fori_loop(0, cht // 4, body, 0)
    swait(2)
    swait(3)
    plsc.subcore_barrier()
    pltpu.sync_copy(acc.at[pl.ds(sid * rpt, rpt)],
                    out_hbm.at[pl.ds(sid * rpt, rpt), pl.ds(col, hw)])

  return agg_kernel


def _make_pair_gather(n_pad, emb, bpt):
  @functools.partial(
      pl.kernel,
      out_type=(
          jax.ShapeDtypeStruct((NW * bpt * 128, emb), jnp.float32),
          jax.ShapeDtypeStruct((NW * bpt * 128, emb), jnp.float32),
      ),
      mesh=_mesh(),
      scratch_types=[
          pltpu.VMEM((bpt * 128,), jnp.int32),
          pltpu.VMEM((bpt * 128,), jnp.int32),
          pltpu.VMEM((128, emb), jnp.float32),
      ],
  )
  def gather_kernel(t_hbm, mid1_hbm, uid1_hbm, outm_hbm, outu_hbm,
                    mid_v, uid_v, rows_v):
    cid = lax.axis_index("c")
    sid = lax.axis_index("s")
    wid = cid * NS + sid
    pltpu.sync_copy(mid1_hbm.at[pl.ds(wid * bpt * 128, bpt * 128)], mid_v)
    pltpu.sync_copy(uid1_hbm.at[pl.ds(wid * bpt * 128, bpt * 128)], uid_v)

    def body(j, carry):
      base = wid * bpt * 128 + j * 128
      pltpu.sync_copy(t_hbm.at[mid_v.at[pl.ds(j * 128, 128)]], rows_v)
      pltpu.sync_copy(rows_v, outm_hbm.at[pl.ds(base, 128)])
      pltpu.sync_copy(t_hbm.at[uid_v.at[pl.ds(j * 128, 128)]], rows_v)
      pltpu.sync_copy(rows_v, outu_hbm.at[pl.ds(base, 128)])
      return carry

    lax.fori_loop(0, bpt, body, 0)

  return gather_kernel


def _make_tc1(n_movies, n_users, n_pad, emb):
  n = n_movies + n_users

  def tc1_body(me_ref, ue_ref, w1t_ref, d0_ref, d1_ref, dis_ref, h1p_ref):
    deg = d0_ref[...] + d1_ref[...] + 1.0
    dis = lax.rsqrt(deg)
    dis_ref[...] = dis
    h1p_ref[0:n_movies, :] = dis[0:n_movies] * jnp.dot(
        me_ref[...], w1t_ref[...], preferred_element_type=jnp.float32)
    h1p_ref[n_movies:n, :] = dis[n_movies:n] * jnp.dot(
        ue_ref[...], w1t_ref[...], preferred_element_type=jnp.float32)
    h1p_ref[n:n_pad, :] = jnp.zeros((n_pad - n, emb), jnp.float32)

  return tc1_body


def _tc2_body(a_ref, h1p_ref, dis_ref, b1_ref, w2t_ref, h2p_ref):
  s = a_ref[...] + h1p_ref[...]
  out1 = jnp.maximum(dis_ref[...] * s + b1_ref[...], 0.0)
  h2p_ref[...] = dis_ref[...] * jnp.dot(out1, w2t_ref[...],
                                        preferred_element_type=jnp.float32)


def _make_tc3(n_movies, n_users, n_pad, emb):
  n = n_movies + n_users

  def tc3_body(a_ref, h2p_ref, dis_ref, b2_ref, wmt_ref, wut_ref, t_ref):
    out2 = dis_ref[...] * (a_ref[...] + h2p_ref[...]) + b2_ref[...]
    t_ref[0:n_movies, :] = jnp.dot(
        out2[0:n_movies, :], wmt_ref[...], preferred_element_type=jnp.float32)
    t_ref[n_movies:n, :] = jnp.dot(
        out2[n_movies:n, :], wut_ref[...], preferred_element_type=jnp.float32)
    t_ref[n:n_pad, :] = jnp.zeros((n_pad - n, emb), jnp.float32)

  return tc3_body


def _tc4_body(rm_ref, ru_ref, g_ref, wgt_ref, fb1_ref, w2_ref, fb2_ref,
              out_ref):
  gpart = jnp.dot(g_ref[...], wgt_ref[...], preferred_element_type=jnp.float32)
  fc1 = jnp.maximum(rm_ref[...] + ru_ref[...] + gpart + fb1_ref[...], 0.0)
  out_ref[...] = jnp.sum(fc1 * w2_ref[...], axis=1, keepdims=True) + fb2_ref[...]


def kernel(movie_id, user_id, genre_id, edge_index, movie_emb, user_emb,
           W1, b1, W2, b2, fcW1, fcb1, fcW2, fcb2):
  f32, i32 = jnp.float32, jnp.int32
  n_movies, emb = movie_emb.shape
  n_users = user_emb.shape[0]
  n = n_movies + n_users
  hid = W1.shape[0]
  bsz = movie_id.shape[0]
  e = edge_index.shape[1]

  n_pad = ((n + 1 + 127) // 128) * 128
  ch = (e + NW * 128 - 1) // (NW * 128)
  ch = ((ch + 7) // 8) * 8
  e_pad = NW * ch * 128
  bpt = bsz // (NW * 128)

  src = edge_index[0].astype(i32)
  dst = edge_index[1].astype(i32)
  pad_src = jnp.full((e_pad - e,), n, i32)
  pad_dst = n + jnp.arange(e_pad - e, dtype=i32) % (n_pad - n)
  src1 = jnp.concatenate([src, pad_src])
  dst1 = jnp.concatenate([dst, pad_dst])
  dst2 = dst1.reshape(NW * ch, 128)

  zeros_deg = jnp.zeros((n_pad, DEGW), f32)
  zeros_blk = jnp.zeros((128, emb // 2), f32)
  ones16 = jnp.ones((128, DEGW), f32)

  w1t = W1.T.astype(f32)
  w2t = W2.T.astype(f32)
  wmt = fcW1[:, :emb].T.astype(f32)
  wut = fcW1[:, emb:2 * emb].T.astype(f32)
  wgt = fcW1[:, 2 * emb:].T.astype(f32)
  b1r = b1.reshape(1, hid).astype(f32)
  b2r = b2.reshape(1, hid).astype(f32)
  fb1 = fcb1.reshape(1, hid).astype(f32)
  w2row = fcW2.reshape(1, hid).astype(f32)
  fb2 = fcb2.reshape(1, 1).astype(f32)
  genre = genre_id.astype(f32)

  mid1 = movie_id.astype(i32)
  uid1 = user_id.astype(i32) + n_movies

  deg_parts = _make_deg(n_pad, ch)(dst2, zeros_deg, ones16)
  d0 = deg_parts[0, :, 0:1]
  d1 = deg_parts[1, :, 0:1]

  dis, h1p = pl.pallas_call(
      _make_tc1(n_movies, n_users, n_pad, emb),
      out_shape=(
          jax.ShapeDtypeStruct((n_pad, 1), f32),
          jax.ShapeDtypeStruct((n_pad, emb), f32),
      ),
  )(movie_emb.astype(f32), user_emb.astype(f32), w1t, d0, d1)

  cht = e_pad // (NS * 128)
  agg = _make_agg(n_pad, emb, cht)

  agg1 = agg(h1p, src1, dst1, zeros_blk)

  h2p = pl.pallas_call(
      _tc2_body,
      out_shape=jax.ShapeDtypeStruct((n_pad, emb), f32),
  )(agg1, h1p, dis, b1r, w2t)

  agg2 = agg(h2p, src1, dst1, zeros_blk)

  t_tab = pl.pallas_call(
      _make_tc3(n_movies, n_users, n_pad, emb),
      out_shape=jax.ShapeDtypeStruct((n_pad, emb), f32),
  )(agg2, h2p, dis, b2r, wmt, wut)

  rows_m, rows_u = _make_pair_gather(n_pad, emb, bpt)(t_tab, mid1, uid1)

  out = pl.pallas_call(
      _tc4_body,
      out_shape=jax.ShapeDtypeStruct((bsz, 1), f32),
  )(rows_m, rows_u, genre, wgt, fb1, w2row, fb2)
  return out
